# baseline (speedup 1.0000x reference)
import functools

import jax
import jax.numpy as jnp
from jax import lax
from jax.experimental import pallas as pl
from jax.experimental.pallas import tpu as pltpu
from jax.experimental.pallas import tpu_sc as plsc

D = 32
N_NEGS = 20
RPB = 420
RPB_PAD = 432
GPB = RPB_PAD // 16
BPC = 2
CH_ROWS = BPC * RPB_PAD
CSIZES = ((0, 128), (128, 128), (256, 128), (384, 48))


def _sc_dots(table_i, table_o, iword_i32, idx_pad):
    B = iword_i32.shape[0]
    info = plsc.get_sparse_core_info()
    NC, NS = info.num_cores, info.num_subcores
    NW = NC * NS
    b_w = B // NW
    n_chunks = b_w // BPC
    assert b_w % BPC == 0 and n_chunks % 2 == 0

    mesh = plsc.VectorSubcoreMesh(core_axis_name="c", subcore_axis_name="s")

    @functools.partial(
        pl.kernel, mesh=mesh,
        compiler_params=pltpu.CompilerParams(
            use_tc_tiling_on_sc=False, needs_layout_passes=False),
        out_type=jax.ShapeDtypeStruct((B, RPB_PAD), jnp.float32),
        scratch_types=[
            pltpu.VMEM((b_w,), jnp.int32),
            pltpu.VMEM((b_w, D), jnp.float32),
            pltpu.VMEM((BPC, RPB_PAD), jnp.int32),
            pltpu.VMEM((BPC, RPB_PAD), jnp.int32),
            pltpu.VMEM((CH_ROWS, D), jnp.float32),
            pltpu.VMEM((CH_ROWS, D), jnp.float32),
            pltpu.VMEM((BPC, RPB_PAD), jnp.float32),
            pltpu.VMEM((BPC, RPB_PAD), jnp.float32),
            pltpu.SemaphoreType.DMA,
            pltpu.SemaphoreType.DMA,
        ],
    )
    def dots_kernel(ti_hbm, to_hbm, iw_hbm, io_hbm, dots_out,
                    iw_v, iv_v, idx_a, idx_b, rows_a, rows_b,
                    dots_a, dots_b, sem_a, sem_b):
        wid = lax.axis_index("s") * NC + lax.axis_index("c")
        wb = wid * b_w

        pltpu.sync_copy(iw_hbm.at[pl.ds(wb, b_w)], iw_v)
        pltpu.make_async_copy(ti_hbm.at[iw_v], iv_v, sem_a).start()
        pltpu.make_async_copy(ti_hbm.at[iw_v], iv_v, sem_a).wait()

        iota16 = lax.iota(jnp.int32, 16)

        def fire(c, idx_v, rows_v, sem):
            pltpu.sync_copy(io_hbm.at[pl.ds(wb + c * BPC, BPC)], idx_v)
            for r in range(BPC):
                for o, sz in CSIZES:
                    pltpu.make_async_copy(
                        to_hbm.at[idx_v.at[r, pl.ds(o, sz)]],
                        rows_v.at[pl.ds(r * RPB_PAD + o, sz)], sem).start()

        def drain(idx_v, rows_v, sem):
            for r in range(BPC):
                for o, sz in CSIZES:
                    pltpu.make_async_copy(
                        to_hbm.at[idx_v.at[r, pl.ds(o, sz)]],
                        rows_v.at[pl.ds(r * RPB_PAD + o, sz)], sem).wait()

        def process(c, rows_v, dots_v):
            for bl in range(BPC):
                iv_lo = iv_v[c * BPC + bl, pl.ds(0, 16)]
                iv_hi = iv_v[c * BPC + bl, pl.ds(16, 16)]

                def grp(g, carry, bl=bl, iv_lo=iv_lo, iv_hi=iv_hi):
                    accv = jnp.zeros((16,), jnp.float32)
                    for j in range(16):
                        r = bl * RPB_PAD + g * 16 + j
                        a = rows_v[r, pl.ds(0, 16)]
                        b = rows_v[r, pl.ds(16, 16)]
                        w = a * iv_lo + b * iv_hi
                        s = plsc.cumsum(w)[15]
                        accv = jnp.where(iota16 == j, s, accv)
                    dots_v[bl, pl.ds(g * 16, 16)] = accv
                    return carry

                lax.fori_loop(0, GPB, grp, 0)

            pltpu.sync_copy(dots_v, dots_out.at[pl.ds(wb + c * BPC, BPC)])

        fire(0, idx_a, rows_a, sem_a)

        def loop(t, carry):
            ca = 2 * t
            fire(ca + 1, idx_b, rows_b, sem_b)
            drain(idx_a, rows_a, sem_a)
            process(ca, rows_a, dots_a)
            fire(lax.rem(ca + 2, n_chunks), idx_a, rows_a, sem_a)
            drain(idx_b, rows_b, sem_b)
            process(ca + 1, rows_b, dots_b)
            return carry

        lax.fori_loop(0, n_chunks // 2, loop, 0)
        drain(idx_a, rows_a, sem_a)

    return dots_kernel(table_i, table_o, iword_i32, idx_pad)


def _tc_loss_sum(dots2d, C):
    B, _ = dots2d.shape

    def body(d_ref, out_ref):
        d = d_ref[...]
        col = lax.broadcasted_iota(jnp.int32, (B, RPB_PAD), 1)
        x = jnp.where(col < C, d, -d)
        ls = jnp.minimum(x, 0.0) - jnp.log(1.0 + jnp.exp(-jnp.abs(x)))
        out_ref[...] = jnp.full(
            (1, 1), jnp.sum(jnp.where(col < RPB, ls, 0.0)), jnp.float32)

    out = pl.pallas_call(
        body,
        out_shape=jax.ShapeDtypeStruct((1, 1), jnp.float32),
    )(dots2d)
    return out[0, 0]


def kernel(iword, owords, table_i, table_o):
    B = iword.shape[0]
    C = owords.shape[1]
    V = table_i.shape[0]

    nwords = jax.random.randint(jax.random.key(1), (B, C * N_NEGS), 0, V - 1)

    idx_pad = jnp.concatenate(
        [owords.astype(jnp.int32), nwords.astype(jnp.int32),
         jnp.zeros((B, RPB_PAD - RPB), jnp.int32)], axis=1)

    dots = _sc_dots(table_i, table_o, iword.astype(jnp.int32), idx_pad)
    total = _tc_loss_sum(dots, C)
    return -total / jnp.float32(B * C)


# speedup vs baseline: 1.3758x; 1.3758x over previous
import functools

import jax
import jax.numpy as jnp
from jax import lax
from jax.experimental import pallas as pl
from jax.experimental.pallas import tpu as pltpu
from jax.experimental.pallas import tpu_sc as plsc

D = 32
N_NEGS = 20
RPB = 420
RPB_PAD = 432
GPB = RPB_PAD // 16
BPC = 2
CH_ROWS = BPC * RPB_PAD
NSIZES = ((0, 128), (128, 128), (256, 128), (384, 16))


def _sc_dots(table_i, table_o, iword_i32, ow_i32, nw_i32):
    B = iword_i32.shape[0]
    C = ow_i32.shape[1]
    info = plsc.get_sparse_core_info()
    NC, NS = info.num_cores, info.num_subcores
    NW = NC * NS
    b_w = B // NW
    n_chunks = b_w // BPC
    assert b_w % BPC == 0 and n_chunks % 2 == 0

    mesh = plsc.VectorSubcoreMesh(core_axis_name="c", subcore_axis_name="s")

    @functools.partial(
        pl.kernel, mesh=mesh,
        compiler_params=pltpu.CompilerParams(
            use_tc_tiling_on_sc=False, needs_layout_passes=False),
        out_type=jax.ShapeDtypeStruct((B, RPB_PAD), jnp.float32),
        scratch_types=[
            pltpu.VMEM((b_w,), jnp.int32),
            pltpu.VMEM((b_w, D), jnp.float32),
            pltpu.VMEM((BPC, 20), jnp.int32),
            pltpu.VMEM((BPC, 20), jnp.int32),
            pltpu.VMEM((BPC, RPB - 20), jnp.int32),
            pltpu.VMEM((BPC, RPB - 20), jnp.int32),
            pltpu.VMEM((CH_ROWS, D), jnp.float32),
            pltpu.VMEM((CH_ROWS, D), jnp.float32),
            pltpu.VMEM((BPC, RPB_PAD), jnp.float32),
            pltpu.VMEM((BPC, RPB_PAD), jnp.float32),
            pltpu.SemaphoreType.DMA,
            pltpu.SemaphoreType.DMA,
        ],
    )
    def dots_kernel(ti_hbm, to_hbm, iw_hbm, ow_hbm, nw_hbm, dots_out,
                    iw_v, iv_v, oi_a, oi_b, ni_a, ni_b, rows_a, rows_b,
                    dots_a, dots_b, sem_a, sem_b):
        wid = lax.axis_index("s") * NC + lax.axis_index("c")
        wb = wid * b_w

        pltpu.sync_copy(iw_hbm.at[pl.ds(wb, b_w)], iw_v)
        pltpu.make_async_copy(ti_hbm.at[iw_v], iv_v, sem_a).start()
        pltpu.make_async_copy(ti_hbm.at[iw_v], iv_v, sem_a).wait()

        iota16 = lax.iota(jnp.int32, 16)

        def dmas(oi_v, ni_v, rows_v, sem):
            out = []
            for r in range(BPC):
                out.append(pltpu.make_async_copy(
                    to_hbm.at[oi_v.at[r, pl.ds(0, C)]],
                    rows_v.at[pl.ds(r * RPB_PAD, C)], sem))
                for o, sz in NSIZES:
                    out.append(pltpu.make_async_copy(
                        to_hbm.at[ni_v.at[r, pl.ds(o, sz)]],
                        rows_v.at[pl.ds(r * RPB_PAD + C + o, sz)], sem))
            return out

        def fire(c, oi_v, ni_v, rows_v, sem):
            pltpu.sync_copy(ow_hbm.at[pl.ds(wb + c * BPC, BPC)], oi_v)
            pltpu.sync_copy(nw_hbm.at[pl.ds(wb + c * BPC, BPC)], ni_v)
            for cp in dmas(oi_v, ni_v, rows_v, sem):
                cp.start()

        def drain(oi_v, ni_v, rows_v, sem):
            for cp in dmas(oi_v, ni_v, rows_v, sem):
                cp.wait()

        def process(c, rows_v, dots_v):
            for bl in range(BPC):
                iv_lo = iv_v[c * BPC + bl, pl.ds(0, 16)]
                iv_hi = iv_v[c * BPC + bl, pl.ds(16, 16)]

                def grp(g, carry, bl=bl, iv_lo=iv_lo, iv_hi=iv_hi):
                    accv = jnp.zeros((16,), jnp.float32)
                    for j in range(16):
                        r = bl * RPB_PAD + g * 16 + j
                        a = rows_v[r, pl.ds(0, 16)]
                        b = rows_v[r, pl.ds(16, 16)]
                        w = a * iv_lo + b * iv_hi
                        s = plsc.cumsum(w)[15]
                        accv = jnp.where(iota16 == j, s, accv)
                    dots_v[bl, pl.ds(g * 16, 16)] = accv
                    return carry

                lax.fori_loop(0, GPB, grp, 0)

            pltpu.sync_copy(dots_v, dots_out.at[pl.ds(wb + c * BPC, BPC)])

        fire(0, oi_a, ni_a, rows_a, sem_a)

        def loop(t, carry):
            ca = 2 * t
            fire(ca + 1, oi_b, ni_b, rows_b, sem_b)
            drain(oi_a, ni_a, rows_a, sem_a)
            process(ca, rows_a, dots_a)
            fire(lax.rem(ca + 2, n_chunks), oi_a, ni_a, rows_a, sem_a)
            drain(oi_b, ni_b, rows_b, sem_b)
            process(ca + 1, rows_b, dots_b)
            return carry

        lax.fori_loop(0, n_chunks // 2, loop, 0)
        drain(oi_a, ni_a, rows_a, sem_a)

    return dots_kernel(table_i, table_o, iword_i32, ow_i32, nw_i32)


def _tc_loss_sum(dots2d, C):
    B, _ = dots2d.shape

    def body(d_ref, out_ref):
        d = d_ref[...]
        col = lax.broadcasted_iota(jnp.int32, (B, RPB_PAD), 1)
        x = jnp.where(col < C, d, -d)
        ls = jnp.minimum(x, 0.0) - jnp.log(1.0 + jnp.exp(-jnp.abs(x)))
        out_ref[...] = jnp.full(
            (1, 1), jnp.sum(jnp.where(col < RPB, ls, 0.0)), jnp.float32)

    out = pl.pallas_call(
        body,
        out_shape=jax.ShapeDtypeStruct((1, 1), jnp.float32),
    )(dots2d)
    return out[0, 0]


def kernel(iword, owords, table_i, table_o):
    B = iword.shape[0]
    C = owords.shape[1]
    V = table_i.shape[0]

    nwords = jax.random.randint(jax.random.key(1), (B, C * N_NEGS), 0, V - 1)

    dots = _sc_dots(table_i, table_o, iword.astype(jnp.int32),
                    owords.astype(jnp.int32), nwords.astype(jnp.int32))
    total = _tc_loss_sum(dots, C)
    return -total / jnp.float32(B * C)


# speedup vs baseline: 1.4133x; 1.0272x over previous
import functools

import jax
import jax.numpy as jnp
from jax import lax
from jax.experimental import pallas as pl
from jax.experimental.pallas import tpu as pltpu
from jax.experimental.pallas import tpu_sc as plsc

D = 32
N_NEGS = 20
RPB = 420
RPB_PAD = 432
GPB = RPB_PAD // 16
BPC = 4
CH_ROWS = BPC * RPB_PAD
NSIZES = ((0, 128), (128, 128), (256, 128), (384, 16))


def _sc_dots(table_i, table_o, iword_i32, ow_i32, nw_i32):
    B = iword_i32.shape[0]
    C = ow_i32.shape[1]
    info = plsc.get_sparse_core_info()
    NC, NS = info.num_cores, info.num_subcores
    NW = NC * NS
    b_w = B // NW
    n_chunks = b_w // BPC
    assert b_w % BPC == 0 and n_chunks % 2 == 0

    mesh = plsc.VectorSubcoreMesh(core_axis_name="c", subcore_axis_name="s")

    @functools.partial(
        pl.kernel, mesh=mesh,
        compiler_params=pltpu.CompilerParams(
            use_tc_tiling_on_sc=False, needs_layout_passes=False),
        out_type=jax.ShapeDtypeStruct((B, RPB_PAD), jnp.float32),
        scratch_types=[
            pltpu.VMEM((b_w,), jnp.int32),
            pltpu.VMEM((b_w, D), jnp.float32),
            pltpu.VMEM((BPC, 20), jnp.int32),
            pltpu.VMEM((BPC, 20), jnp.int32),
            pltpu.VMEM((BPC, RPB - 20), jnp.int32),
            pltpu.VMEM((BPC, RPB - 20), jnp.int32),
            pltpu.VMEM((CH_ROWS, D), jnp.float32),
            pltpu.VMEM((CH_ROWS, D), jnp.float32),
            pltpu.VMEM((BPC, RPB_PAD), jnp.float32),
            pltpu.VMEM((BPC, RPB_PAD), jnp.float32),
            pltpu.SemaphoreType.DMA,
            pltpu.SemaphoreType.DMA,
        ],
    )
    def dots_kernel(ti_hbm, to_hbm, iw_hbm, ow_hbm, nw_hbm, dots_out,
                    iw_v, iv_v, oi_a, oi_b, ni_a, ni_b, rows_a, rows_b,
                    dots_a, dots_b, sem_a, sem_b):
        wid = lax.axis_index("s") * NC + lax.axis_index("c")
        wb = wid * b_w

        pltpu.sync_copy(iw_hbm.at[pl.ds(wb, b_w)], iw_v)
        pltpu.make_async_copy(ti_hbm.at[iw_v], iv_v, sem_a).start()
        pltpu.make_async_copy(ti_hbm.at[iw_v], iv_v, sem_a).wait()

        iota16 = lax.iota(jnp.int32, 16)

        def dmas(oi_v, ni_v, rows_v, sem):
            out = []
            for r in range(BPC):
                out.append(pltpu.make_async_copy(
                    to_hbm.at[oi_v.at[r, pl.ds(0, C)]],
                    rows_v.at[pl.ds(r * RPB_PAD, C)], sem))
                for o, sz in NSIZES:
                    out.append(pltpu.make_async_copy(
                        to_hbm.at[ni_v.at[r, pl.ds(o, sz)]],
                        rows_v.at[pl.ds(r * RPB_PAD + C + o, sz)], sem))
            return out

        def fire(c, oi_v, ni_v, rows_v, sem):
            pltpu.sync_copy(ow_hbm.at[pl.ds(wb + c * BPC, BPC)], oi_v)
            pltpu.sync_copy(nw_hbm.at[pl.ds(wb + c * BPC, BPC)], ni_v)
            for cp in dmas(oi_v, ni_v, rows_v, sem):
                cp.start()

        def drain(oi_v, ni_v, rows_v, sem):
            for cp in dmas(oi_v, ni_v, rows_v, sem):
                cp.wait()

        def process(c, rows_v, dots_v):
            for bl in range(BPC):
                iv_lo = iv_v[c * BPC + bl, pl.ds(0, 16)]
                iv_hi = iv_v[c * BPC + bl, pl.ds(16, 16)]

                def grp(g, carry, bl=bl, iv_lo=iv_lo, iv_hi=iv_hi):
                    accv = jnp.zeros((16,), jnp.float32)
                    for j in range(16):
                        r = bl * RPB_PAD + g * 16 + j
                        a = rows_v[r, pl.ds(0, 16)]
                        b = rows_v[r, pl.ds(16, 16)]
                        w = a * iv_lo + b * iv_hi
                        s = plsc.cumsum(w)[15]
                        accv = jnp.where(iota16 == j, s, accv)
                    dots_v[bl, pl.ds(g * 16, 16)] = accv
                    return carry

                lax.fori_loop(0, GPB, grp, 0)

            pltpu.sync_copy(dots_v, dots_out.at[pl.ds(wb + c * BPC, BPC)])

        fire(0, oi_a, ni_a, rows_a, sem_a)

        def loop(t, carry):
            ca = 2 * t
            fire(ca + 1, oi_b, ni_b, rows_b, sem_b)
            drain(oi_a, ni_a, rows_a, sem_a)
            process(ca, rows_a, dots_a)
            fire(lax.rem(ca + 2, n_chunks), oi_a, ni_a, rows_a, sem_a)
            drain(oi_b, ni_b, rows_b, sem_b)
            process(ca + 1, rows_b, dots_b)
            return carry

        lax.fori_loop(0, n_chunks // 2, loop, 0)
        drain(oi_a, ni_a, rows_a, sem_a)

    return dots_kernel(table_i, table_o, iword_i32, ow_i32, nw_i32)


def _tc_loss_sum(dots2d, C):
    B, _ = dots2d.shape

    def body(d_ref, out_ref):
        d = d_ref[...]
        col = lax.broadcasted_iota(jnp.int32, (B, RPB_PAD), 1)
        x = jnp.where(col < C, d, -d)
        ls = jnp.minimum(x, 0.0) - jnp.log(1.0 + jnp.exp(-jnp.abs(x)))
        out_ref[...] = jnp.full(
            (1, 1), jnp.sum(jnp.where(col < RPB, ls, 0.0)), jnp.float32)

    out = pl.pallas_call(
        body,
        out_shape=jax.ShapeDtypeStruct((1, 1), jnp.float32),
    )(dots2d)
    return out[0, 0]


def kernel(iword, owords, table_i, table_o):
    B = iword.shape[0]
    C = owords.shape[1]
    V = table_i.shape[0]

    nwords = jax.random.randint(jax.random.key(1), (B, C * N_NEGS), 0, V - 1)

    dots = _sc_dots(table_i, table_o, iword.astype(jnp.int32),
                    owords.astype(jnp.int32), nwords.astype(jnp.int32))
    total = _tc_loss_sum(dots, C)
    return -total / jnp.float32(B * C)


# speedup vs baseline: 1.4152x; 1.0013x over previous
import functools

import jax
import jax.numpy as jnp
from jax import lax
from jax.experimental import pallas as pl
from jax.experimental.pallas import tpu as pltpu
from jax.experimental.pallas import tpu_sc as plsc

D = 32
N_NEGS = 20
RPB = 420
RPB_PAD = 432
GPB = RPB_PAD // 16
BPC = 4
CH_ROWS = BPC * RPB_PAD
NSIZES = ((0, 128), (128, 128), (256, 128), (384, 16))


def _sc_dots(table_i, table_o, iword_i32, ow_i32, nw_i32):
    B = iword_i32.shape[0]
    C = ow_i32.shape[1]
    info = plsc.get_sparse_core_info()
    NC, NS = info.num_cores, info.num_subcores
    NW = NC * NS
    b_w = B // NW
    n_chunks = b_w // BPC
    assert b_w % BPC == 0 and n_chunks % 2 == 0

    mesh = plsc.VectorSubcoreMesh(core_axis_name="c", subcore_axis_name="s")

    @functools.partial(
        pl.kernel, mesh=mesh,
        compiler_params=pltpu.CompilerParams(
            use_tc_tiling_on_sc=False, needs_layout_passes=False),
        out_type=jax.ShapeDtypeStruct((B, RPB_PAD), jnp.float32),
        scratch_types=[
            pltpu.VMEM((b_w,), jnp.int32),
            pltpu.VMEM((b_w, D), jnp.float32),
            pltpu.VMEM((BPC, 20), jnp.int32),
            pltpu.VMEM((BPC, 20), jnp.int32),
            pltpu.VMEM((BPC, RPB - 20), jnp.int32),
            pltpu.VMEM((BPC, RPB - 20), jnp.int32),
            pltpu.VMEM((CH_ROWS, D), jnp.float32),
            pltpu.VMEM((CH_ROWS, D), jnp.float32),
            pltpu.VMEM((BPC, RPB_PAD), jnp.float32),
            pltpu.VMEM((BPC, RPB_PAD), jnp.float32),
            pltpu.SemaphoreType.DMA,
            pltpu.SemaphoreType.DMA,
        ],
    )
    def dots_kernel(ti_hbm, to_hbm, iw_hbm, ow_hbm, nw_hbm, dots_out,
                    iw_v, iv_v, oi_a, oi_b, ni_a, ni_b, rows_a, rows_b,
                    dots_a, dots_b, sem_a, sem_b):
        wid = lax.axis_index("s") * NC + lax.axis_index("c")
        wb = wid * b_w

        pltpu.sync_copy(iw_hbm.at[pl.ds(wb, b_w)], iw_v)
        pltpu.make_async_copy(ti_hbm.at[iw_v], iv_v, sem_a).start()
        pltpu.make_async_copy(ti_hbm.at[iw_v], iv_v, sem_a).wait()

        iota16 = lax.iota(jnp.int32, 16)

        def dmas(oi_v, ni_v, rows_v, sem):
            out = []
            for r in range(BPC):
                out.append(pltpu.make_async_copy(
                    to_hbm.at[oi_v.at[r, pl.ds(0, C)]],
                    rows_v.at[pl.ds(r * RPB_PAD, C)], sem))
                for o, sz in NSIZES:
                    out.append(pltpu.make_async_copy(
                        to_hbm.at[ni_v.at[r, pl.ds(o, sz)]],
                        rows_v.at[pl.ds(r * RPB_PAD + C + o, sz)], sem))
            return out

        def fire(c, oi_v, ni_v, rows_v, sem):
            pltpu.sync_copy(ow_hbm.at[pl.ds(wb + c * BPC, BPC)], oi_v)
            pltpu.sync_copy(nw_hbm.at[pl.ds(wb + c * BPC, BPC)], ni_v)
            for cp in dmas(oi_v, ni_v, rows_v, sem):
                cp.start()

        def drain(oi_v, ni_v, rows_v, sem):
            for cp in dmas(oi_v, ni_v, rows_v, sem):
                cp.wait()

        def process(c, rows_v, dots_v):
            for bl in range(BPC):
                iv_lo = iv_v[c * BPC + bl, pl.ds(0, 16)]
                iv_hi = iv_v[c * BPC + bl, pl.ds(16, 16)]

                def grp(g, carry, bl=bl, iv_lo=iv_lo, iv_hi=iv_hi):
                    accv = jnp.zeros((16,), jnp.float32)
                    for j in range(16):
                        r = bl * RPB_PAD + g * 16 + j
                        a = rows_v[r, pl.ds(0, 16)]
                        b = rows_v[r, pl.ds(16, 16)]
                        w = a * iv_lo + b * iv_hi
                        s = plsc.cumsum(w)[15]
                        accv = jnp.where(iota16 == j, s, accv)
                    dots_v[bl, pl.ds(g * 16, 16)] = accv
                    return carry

                lax.fori_loop(0, GPB, grp, 0)

            pltpu.sync_copy(dots_v, dots_out.at[pl.ds(wb + c * BPC, BPC)])

        fire(0, oi_a, ni_a, rows_a, sem_a)

        def loop(t, carry):
            ca = 2 * t
            fire(ca + 1, oi_b, ni_b, rows_b, sem_b)
            drain(oi_a, ni_a, rows_a, sem_a)
            process(ca, rows_a, dots_a)

            @pl.when(ca + 2 < n_chunks)
            def _():
                fire(ca + 2, oi_a, ni_a, rows_a, sem_a)

            drain(oi_b, ni_b, rows_b, sem_b)
            process(ca + 1, rows_b, dots_b)
            return carry

        lax.fori_loop(0, n_chunks // 2, loop, 0)

    return dots_kernel(table_i, table_o, iword_i32, ow_i32, nw_i32)


def _tc_loss_sum(dots2d, C):
    B, _ = dots2d.shape

    def body(d_ref, out_ref):
        d = d_ref[...]
        col = lax.broadcasted_iota(jnp.int32, (B, RPB_PAD), 1)
        x = jnp.where(col < C, d, -d)
        ls = jnp.minimum(x, 0.0) - jnp.log(1.0 + jnp.exp(-jnp.abs(x)))
        out_ref[...] = jnp.full(
            (1, 1), jnp.sum(jnp.where(col < RPB, ls, 0.0)), jnp.float32)

    out = pl.pallas_call(
        body,
        out_shape=jax.ShapeDtypeStruct((1, 1), jnp.float32),
    )(dots2d)
    return out[0, 0]


def kernel(iword, owords, table_i, table_o):
    B = iword.shape[0]
    C = owords.shape[1]
    V = table_i.shape[0]

    nwords = jax.random.randint(jax.random.key(1), (B, C * N_NEGS), 0, V - 1)

    dots = _sc_dots(table_i, table_o, iword.astype(jnp.int32),
                    owords.astype(jnp.int32), nwords.astype(jnp.int32))
    total = _tc_loss_sum(dots, C)
    return -total / jnp.float32(B * C)
